# Initial kernel scaffold; baseline (speedup 1.0000x reference)
import jax
import jax.numpy as jnp
from jax.experimental import pallas as pl


def kernel(x, neighbor_idx, W, b):
    raise NotImplementedError("write your pallas kernel here")


# speedup vs baseline: 1.7589x; 1.7589x over previous
import functools

import jax
import jax.numpy as jnp
from jax import lax
from jax.experimental import pallas as pl
from jax.experimental.pallas import tpu as pltpu
from jax.experimental.pallas import tpu_sc as plsc

D = 128
K = 4
NC, NS = 2, 16
NW = NC * NS
CH = 128


def _mm_body(x_ref, w0_ref, wn_ref, b_ref, y0_ref, z1_ref, z2_ref, z3_ref, z4_ref):
    xb = x_ref[...]
    y0_ref[...] = (
        jnp.dot(xb, w0_ref[...], preferred_element_type=jnp.float32) + b_ref[...]
    )
    zn = jnp.dot(xb, wn_ref[...], preferred_element_type=jnp.float32)
    z1_ref[...] = zn[:, 0 * D:1 * D]
    z2_ref[...] = zn[:, 1 * D:2 * D]
    z3_ref[...] = zn[:, 2 * D:3 * D]
    z4_ref[...] = zn[:, 3 * D:4 * D]


def _make_sc_kernel(np_rows: int):
    bpw = np_rows // NW
    nch = bpw // CH
    mesh = plsc.VectorSubcoreMesh(
        core_axis_name="c", subcore_axis_name="s", num_cores=NC, num_subcores=NS
    )

    @functools.partial(
        pl.kernel,
        out_type=jax.ShapeDtypeStruct((np_rows, D), jnp.float32),
        mesh=mesh,
        scratch_types=[
            pltpu.VMEM((K, CH), jnp.int32),
            pltpu.VMEM((CH, D), jnp.float32),
            pltpu.VMEM((K, CH, D), jnp.float32),
            pltpu.SemaphoreType.DMA,
        ],
    )
    def sc_gather_sum(y0_hbm, z1_hbm, z2_hbm, z3_hbm, z4_hbm, idxt_hbm, out_hbm,
                      idx_s, acc_s, g_s, sem):
        wid = lax.axis_index("s") * NC + lax.axis_index("c")
        wbase = wid * bpw
        z_hbms = (z1_hbm, z2_hbm, z3_hbm, z4_hbm)

        def chunk_body(c, carry):
            base = pl.multiple_of(wbase + c * CH, CH)
            pltpu.sync_copy(idxt_hbm.at[:, pl.ds(base, CH)], idx_s)
            cp_y = pltpu.async_copy(y0_hbm.at[pl.ds(base, CH)], acc_s, sem)
            cps = [
                pltpu.async_copy(z_hbms[k].at[idx_s.at[k]], g_s.at[k], sem)
                for k in range(K)
            ]
            cp_y.wait()
            for cp in cps:
                cp.wait()

            def row_body(i, rcarry):
                for j in range(D // 16):
                    sl = pl.ds(j * 16, 16)
                    acc_s[i, sl] = (
                        acc_s[i, sl]
                        + g_s[0, i, sl] + g_s[1, i, sl]
                        + g_s[2, i, sl] + g_s[3, i, sl]
                    )
                return rcarry

            lax.fori_loop(0, CH, row_body, 0, unroll=False)
            pltpu.sync_copy(acc_s, out_hbm.at[pl.ds(base, CH)])
            return carry

        lax.fori_loop(0, nch, chunk_body, 0, unroll=False)

    return sc_gather_sum


def kernel(x, neighbor_idx, W, b):
    n, d = x.shape
    assert d == D
    np_rows = ((n + NW * CH - 1) // (NW * CH)) * (NW * CH)
    blk = 2048

    xp = jnp.pad(x, ((0, np_rows - n), (0, 0)))
    idxt = jnp.pad(
        neighbor_idx.astype(jnp.int32).T, ((0, 0), (0, np_rows - n))
    )
    wt = W.T
    w0 = wt[0:D]
    wn = jnp.concatenate([wt[(k + 1) * D:(k + 2) * D] for k in range(K)], axis=1)
    b2 = b.reshape(1, D)

    mm = pl.pallas_call(
        _mm_body,
        grid=(np_rows // blk,),
        in_specs=[
            pl.BlockSpec((blk, D), lambda i: (i, 0)),
            pl.BlockSpec((D, D), lambda i: (0, 0)),
            pl.BlockSpec((D, K * D), lambda i: (0, 0)),
            pl.BlockSpec((1, D), lambda i: (0, 0)),
        ],
        out_specs=[pl.BlockSpec((blk, D), lambda i: (i, 0)) for _ in range(5)],
        out_shape=[jax.ShapeDtypeStruct((np_rows, D), jnp.float32) for _ in range(5)],
    )
    y0, z1, z2, z3, z4 = mm(xp, w0, wn, b2)

    out_p = _make_sc_kernel(np_rows)(y0, z1, z2, z3, z4, idxt)
    return out_p[:n]


# speedup vs baseline: 2.8795x; 1.6371x over previous
import functools

import jax
import jax.numpy as jnp
from jax import lax
from jax.experimental import pallas as pl
from jax.experimental.pallas import tpu as pltpu
from jax.experimental.pallas import tpu_sc as plsc

D = 128
K = 4
NC, NS = 2, 16
NW = NC * NS
CH = 56


def _mm_body(x_ref, w0_ref, wn_ref, b_ref, y0_ref, z1_ref, z2_ref, z3_ref, z4_ref):
    xb = x_ref[...]
    y0_ref[...] = (
        jnp.dot(xb, w0_ref[...], preferred_element_type=jnp.float32) + b_ref[...]
    )
    zn = jnp.dot(xb, wn_ref[...], preferred_element_type=jnp.float32)
    z1_ref[...] = zn[:, 0 * D:1 * D]
    z2_ref[...] = zn[:, 1 * D:2 * D]
    z3_ref[...] = zn[:, 2 * D:3 * D]
    z4_ref[...] = zn[:, 3 * D:4 * D]


def _make_sc_kernel(np_rows: int):
    bpw = np_rows // NW
    nch = bpw // CH
    mesh = plsc.VectorSubcoreMesh(
        core_axis_name="c", subcore_axis_name="s", num_cores=NC, num_subcores=NS
    )

    @functools.partial(
        pl.kernel,
        out_type=jax.ShapeDtypeStruct((np_rows, D), jnp.float32),
        mesh=mesh,
        scratch_types=[
            pltpu.VMEM((2, K, CH), jnp.int32),
            pltpu.VMEM((2, CH, D), jnp.float32),
            pltpu.VMEM((2, K, CH, D), jnp.float32),
            pltpu.SemaphoreType.DMA,
            pltpu.SemaphoreType.DMA,
            pltpu.SemaphoreType.DMA,
            pltpu.SemaphoreType.DMA,
        ],
    )
    def sc_gather_sum(y0_hbm, z1_hbm, z2_hbm, z3_hbm, z4_hbm, idxf_hbm, out_hbm,
                      idx_s, acc_s, g_s, sem_g0, sem_g1, sem_o0, sem_o1):
        wid = lax.axis_index("s") * NC + lax.axis_index("c")
        wbase = wid * bpw
        z_hbms = (z1_hbm, z2_hbm, z3_hbm, z4_hbm)
        sem_g = (sem_g0, sem_g1)
        sem_o = (sem_o0, sem_o1)

        def in_copies(c, p):
            base = wbase + c * CH
            cps = [
                pltpu.make_async_copy(
                    y0_hbm.at[pl.ds(base, CH)], acc_s.at[p], sem_g[p]
                )
            ]
            for k in range(K):
                cps.append(
                    pltpu.make_async_copy(
                        z_hbms[k].at[idx_s.at[p, k]], g_s.at[p, k], sem_g[p]
                    )
                )
            return cps

        def out_copy(c, p):
            base = wbase + c * CH
            return pltpu.make_async_copy(
                acc_s.at[p], out_hbm.at[pl.ds(base, CH)], sem_o[p]
            )

        def stage_and_fire(c, p):
            base = wbase + c * CH
            for k in range(K):
                pltpu.sync_copy(
                    idxf_hbm.at[pl.ds(k * np_rows + base, CH)], idx_s.at[p, k]
                )
            for cp in in_copies(c, p):
                cp.start()

        def chunk_step(c, p):
            q = 1 - p

            @pl.when(c + 1 < nch)
            def _fire_next():
                @pl.when(c >= 1)
                def _drain_prev_out():
                    out_copy(c - 1, q).wait()

                stage_and_fire(c + 1, q)

            for cp in in_copies(c, p):
                cp.wait()

            def row_body(i, rcarry):
                for j in range(D // 16):
                    sl = pl.ds(j * 16, 16)
                    acc_s[p, i, sl] = (
                        acc_s[p, i, sl]
                        + g_s[p, 0, i, sl] + g_s[p, 1, i, sl]
                        + g_s[p, 2, i, sl] + g_s[p, 3, i, sl]
                    )
                return rcarry

            lax.fori_loop(0, CH, row_body, 0)
            out_copy(c, p).start()

        stage_and_fire(0, 0)

        def pair_body(t, carry):
            chunk_step(2 * t, 0)
            chunk_step(2 * t + 1, 1)
            return carry

        assert nch % 2 == 0
        lax.fori_loop(0, nch // 2, pair_body, 0)

        out_copy(nch - 2, (nch - 2) % 2).wait()
        out_copy(nch - 1, (nch - 1) % 2).wait()

    return sc_gather_sum


def kernel(x, neighbor_idx, W, b):
    n, d = x.shape
    assert d == D
    np_rows = ((n + NW * CH - 1) // (NW * CH)) * (NW * CH)
    blk = 2048

    xp = jnp.pad(x, ((0, np_rows - n), (0, 0)))
    idxf = jnp.pad(
        neighbor_idx.astype(jnp.int32).T, ((0, 0), (0, np_rows - n))
    ).reshape(K * np_rows)
    wt = W.T
    w0 = wt[0:D]
    wn = jnp.concatenate([wt[(k + 1) * D:(k + 2) * D] for k in range(K)], axis=1)
    b2 = b.reshape(1, D)

    mm = pl.pallas_call(
        _mm_body,
        grid=(np_rows // blk,),
        in_specs=[
            pl.BlockSpec((blk, D), lambda i: (i, 0)),
            pl.BlockSpec((D, D), lambda i: (0, 0)),
            pl.BlockSpec((D, K * D), lambda i: (0, 0)),
            pl.BlockSpec((1, D), lambda i: (0, 0)),
        ],
        out_specs=[pl.BlockSpec((blk, D), lambda i: (i, 0)) for _ in range(5)],
        out_shape=[jax.ShapeDtypeStruct((np_rows, D), jnp.float32) for _ in range(5)],
    )
    y0, z1, z2, z3, z4 = mm(xp, w0, wn, b2)

    out_p = _make_sc_kernel(np_rows)(y0, z1, z2, z3, z4, idxf)
    return out_p[:n]
